# baseline (speedup 1.0000x reference)
import jax
import jax.numpy as jnp
from jax.experimental import pallas as pl

_MASK_TOKEN_ID = 8191
_P = 1024
_K = 16


def _stats_kernel(x_ref, pmax_ref, tok_ref):
    xb = x_ref[0]
    m = jnp.max(xb, axis=1, keepdims=True)
    e = jnp.exp(xb - m)
    s = jnp.sum(e, axis=1, keepdims=True)
    a = jnp.argmax(xb, axis=1).astype(jnp.int32)
    pmax_ref[0] = 1.0 / s
    tok_ref[0] = a[:, None]


def _topk_kernel(pmax_ref, tok_ref, shift_ref, probs_ref, suf_ref):
    p = pmax_ref[...]
    tok = tok_ref[...]
    shift = shift_ref[0, 0]
    b, ssuf = p.shape
    iota = jax.lax.broadcasted_iota(jnp.int32, p.shape, 1)
    colk = jax.lax.broadcasted_iota(jnp.int32, (b, _K), 1)
    sel = jnp.zeros((b, _K), jnp.int32)
    vals = jnp.zeros((b, _K), jnp.float32)
    for i in range(_K):
        m = jnp.max(p, axis=1, keepdims=True)
        cand = jnp.where(p == m, iota, ssuf)
        idx = jnp.min(cand, axis=1, keepdims=True)
        sel = jnp.where(colk == i, idx, sel)
        vals = jnp.where(colk == i, m, vals)
        p = jnp.where(iota == idx, -jnp.inf, p)
    probs_ref[...] = vals
    q = sel + shift
    rank = jnp.zeros_like(q)
    for j in range(_K):
        rank = rank + (q[:, j : j + 1] < q).astype(jnp.int32)
    out = jnp.full(p.shape, _MASK_TOKEN_ID, jnp.int32)
    for i in range(_K):
        pos = q[:, i : i + 1]
        t = jnp.sum(jnp.where(iota == pos, tok, 0), axis=1, keepdims=True)
        out = jnp.where(iota == rank[:, i : i + 1], t, out)
    suf_ref[...] = out


def kernel(logits, x, output_start_idx, k):
    b, s, v = logits.shape
    ssuf = s - _P
    cs = 512
    pmax, tok = pl.pallas_call(
        _stats_kernel,
        grid=(b, ssuf // cs),
        in_specs=[pl.BlockSpec((1, cs, v), lambda i, c: (i, c + _P // cs, 0))],
        out_specs=[
            pl.BlockSpec((1, cs, 1), lambda i, c: (i, c, 0)),
            pl.BlockSpec((1, cs, 1), lambda i, c: (i, c, 0)),
        ],
        out_shape=[
            jax.ShapeDtypeStruct((b, ssuf, 1), jnp.float32),
            jax.ShapeDtypeStruct((b, ssuf, 1), jnp.int32),
        ],
    )(logits)
    pmax2 = pmax.reshape(b, ssuf)
    tok2 = tok.reshape(b, ssuf)
    shift = (jnp.asarray(output_start_idx, jnp.int32) - _P
             + jnp.asarray(k, jnp.int32) - _K).reshape(1, 1)
    probs, out_suf = pl.pallas_call(
        _topk_kernel,
        out_shape=[
            jax.ShapeDtypeStruct((b, _K), jnp.float32),
            jax.ShapeDtypeStruct((b, ssuf), jnp.int32),
        ],
    )(pmax2, tok2, shift)
    out = jnp.concatenate([x[:, :_P], out_suf], axis=1)
    return out, probs


# speedup vs baseline: 1.0171x; 1.0171x over previous
import jax
import jax.numpy as jnp
from jax.experimental import pallas as pl
from jax.experimental.pallas import tpu as pltpu
from jax.experimental.pallas import tpu_sc as plsc

_MASK_TOKEN_ID = 8191
_P = 1024
_K = 16
_L = 16


def _sc_topk_kernel(pmax_hbm, tok_hbm, shift_hbm, out_hbm, probs_hbm,
                    pmax_v, tok_v, shift_v, row_v, probs_v, red_f, red_i):
    wid = jax.lax.axis_index("s") * 2 + jax.lax.axis_index("c")
    row = wid & 7
    pltpu.sync_copy(pmax_hbm.at[row], pmax_v)
    pltpu.sync_copy(tok_hbm.at[row], tok_v)
    pltpu.sync_copy(shift_hbm, shift_v)
    iota = jax.lax.iota(jnp.int32, _L)
    neg = jnp.full((_L,), -jnp.inf, jnp.float32)
    nvec = 1024 // _L
    sel = jnp.zeros((_L,), jnp.int32)
    vals = jnp.zeros((_L,), jnp.float32)

    def _allred(vv, scratch, op):
        for d in (8, 4, 2, 1):
            scratch[...] = vv
            vv = op(vv, plsc.load_gather(scratch, [iota ^ d]))
        return vv

    for i in range(_K):
        def scan(j, carry):
            m, bj = carry
            xv = pmax_v[pl.ds(j * _L, _L)]
            upd = xv > m
            return jnp.maximum(m, xv), jnp.where(upd, j, bj)

        m, bj = jax.lax.fori_loop(
            0, nvec, scan,
            (jnp.full((_L,), -jnp.inf, jnp.float32),
             jnp.zeros((_L,), jnp.int32)))
        gmax = _allred(m, red_f, jnp.maximum)
        cand = jnp.where(m == gmax, bj * _L + iota, nvec * _L)
        bidx = _allred(cand, red_i, jnp.minimum)
        sel = jnp.where(iota == i, bidx, sel)
        vals = jnp.where(iota == i, gmax, vals)
        plsc.store_scatter(pmax_v, [bidx], neg, mask=iota == 0)
    q = sel + shift_v[...]
    tv = plsc.load_gather(tok_v, [q])
    _, tvs = plsc.sort_key_val(q, tv)

    def fill(j, _):
        row_v[pl.ds(j * _L, _L)] = jnp.full((_L,), _MASK_TOKEN_ID, jnp.int32)
        return 0

    jax.lax.fori_loop(0, nvec, fill, 0)
    row_v[pl.ds(0, _L)] = tvs
    probs_v[...] = vals

    @pl.when(wid < 8)
    def _():
        pltpu.sync_copy(row_v, out_hbm.at[row])
        pltpu.sync_copy(probs_v, probs_hbm.at[row])


def _stats_kernel(x_ref, pmax_ref, tok_ref):
    xb = x_ref[0]
    m = jnp.max(xb, axis=1, keepdims=True)
    e = jnp.exp(xb - m)
    s = jnp.sum(e, axis=1, keepdims=True)
    iota = jax.lax.broadcasted_iota(jnp.int32, xb.shape, 1)
    a = jnp.min(jnp.where(xb == m, iota, xb.shape[1]), axis=1, keepdims=True)
    pmax_ref[0] = 1.0 / s
    tok_ref[0] = a


def kernel(logits, x, output_start_idx, k):
    b, s, v = logits.shape
    ssuf = s - _P
    cs = 512
    pmax, tok = pl.pallas_call(
        _stats_kernel,
        grid=(b, ssuf // cs),
        in_specs=[pl.BlockSpec((1, cs, v), lambda i, c: (i, c + _P // cs, 0))],
        out_specs=[
            pl.BlockSpec((1, cs, 1), lambda i, c: (i, c, 0)),
            pl.BlockSpec((1, cs, 1), lambda i, c: (i, c, 0)),
        ],
        out_shape=[
            jax.ShapeDtypeStruct((b, ssuf, 1), jnp.float32),
            jax.ShapeDtypeStruct((b, ssuf, 1), jnp.int32),
        ],
    )(logits)
    pmax2 = pmax.reshape(b, ssuf)
    tok2 = tok.reshape(b, ssuf)
    shift16 = jnp.full((_L,), jnp.asarray(output_start_idx, jnp.int32) - _P
                       + jnp.asarray(k, jnp.int32) - _K, jnp.int32)
    sc_topk = pl.kernel(
        _sc_topk_kernel,
        out_type=[
            jax.ShapeDtypeStruct((b, ssuf), jnp.int32),
            jax.ShapeDtypeStruct((b, _K), jnp.float32),
        ],
        scratch_types=[
            pltpu.VMEM((ssuf,), jnp.float32),
            pltpu.VMEM((ssuf,), jnp.int32),
            pltpu.VMEM((_L,), jnp.int32),
            pltpu.VMEM((ssuf,), jnp.int32),
            pltpu.VMEM((_K,), jnp.float32),
            pltpu.VMEM((_L,), jnp.float32),
            pltpu.VMEM((_L,), jnp.int32),
        ],
        mesh=plsc.VectorSubcoreMesh(core_axis_name="c", subcore_axis_name="s"),
        compiler_params=pltpu.CompilerParams(needs_layout_passes=False),
    )
    out_suf, probs = sc_topk(pmax2, tok2, shift16)
    out = jnp.concatenate([x[:, :_P], out_suf], axis=1)
    return out, probs
